# Initial kernel scaffold; baseline (speedup 1.0000x reference)
import jax
import jax.numpy as jnp
from jax.experimental import pallas as pl


def kernel(x, W1, b1, W2, b2):
    raise NotImplementedError("write your pallas kernel here")


# speedup vs baseline: 1.0537x; 1.0537x over previous
import functools

import jax
import jax.numpy as jnp
from jax.experimental import pallas as pl
from jax.experimental.pallas import tpu as pltpu

_TRUNC = 32
_NEG_SENTINEL = -1.0


def _topk_kernel(x_ref, z_ref):
    x = x_ref[...]
    b, n = x.shape
    col = jax.lax.broadcasted_iota(jnp.int32, (b, n), 1)

    def body(_, carry):
        xm, zlog, sel, minv = carry
        rowmax = jnp.max(xm, axis=1, keepdims=True)
        logv = jnp.clip(jnp.log(rowmax), -1000.0, None) + 50.0
        minv = jnp.minimum(minv, logv)
        poscand = jnp.where(xm == rowmax, col, n)
        firstpos = jnp.min(poscand, axis=1, keepdims=True)
        mask = col == firstpos
        zlog = zlog + jnp.where(mask, logv, 0.0)
        sel = sel + jnp.where(mask, 1.0, 0.0)
        xm = jnp.where(mask, _NEG_SENTINEL, xm)
        return xm, zlog, sel, minv

    zeros = jnp.zeros((b, n), jnp.float32)
    minv0 = jnp.full((b, 1), 1e30, jnp.float32)
    _, zlog, sel, minv = jax.lax.fori_loop(0, _TRUNC, body, (x, zeros, zeros, minv0))
    shift = jax.nn.relu(-minv)
    z = sel * (zlog + shift)
    norm = jnp.sqrt(jnp.sum(z * z, axis=1, keepdims=True))
    z_ref[...] = z / jnp.clip(norm, 1e-12, None)


def _mlp_kernel(z_ref, w1_ref, b1_ref, w2_ref, b2_ref, out_ref, acc_ref):
    j = pl.program_id(0)

    @pl.when(j == 0)
    def _():
        acc_ref[...] = jnp.zeros_like(acc_ref)

    h = jax.lax.dot_general(
        z_ref[...], w1_ref[...], (((1,), (1,)), ((), ())),
        preferred_element_type=jnp.float32) + b1_ref[...]
    h = jnp.where(h >= 0, h, 0.2 * h)
    acc_ref[...] += jax.lax.dot_general(
        h, w2_ref[...], (((1,), (1,)), ((), ())),
        preferred_element_type=jnp.float32)

    @pl.when(j == pl.num_programs(0) - 1)
    def _():
        out_ref[...] = acc_ref[...] + b2_ref[...]


@functools.partial(jax.jit, static_argnames=("interpret",))
def _impl(x, W1, b1, W2, b2, interpret=False):
    batch, n = x.shape
    out_dim = W2.shape[0]
    z = pl.pallas_call(
        _topk_kernel,
        out_shape=jax.ShapeDtypeStruct((batch, n), jnp.float32),
        interpret=interpret,
    )(x)

    blk = 512
    grid = n // blk
    out = pl.pallas_call(
        _mlp_kernel,
        grid=(grid,),
        in_specs=[
            pl.BlockSpec((batch, n), lambda j: (0, 0)),
            pl.BlockSpec((blk, n), lambda j: (j, 0)),
            pl.BlockSpec((1, blk), lambda j: (0, j)),
            pl.BlockSpec((out_dim, blk), lambda j: (0, j)),
            pl.BlockSpec((1, out_dim), lambda j: (0, 0)),
        ],
        out_specs=pl.BlockSpec((batch, out_dim), lambda j: (0, 0)),
        out_shape=jax.ShapeDtypeStruct((batch, out_dim), jnp.float32),
        scratch_shapes=[pltpu.VMEM((batch, out_dim), jnp.float32)],
        interpret=interpret,
    )(z, W1, b1.reshape(1, -1), W2, b2.reshape(1, -1))
    return out


def kernel(x, W1, b1, W2, b2):
    return _impl(x, W1, b1, W2, b2)
